# Initial kernel scaffold; baseline (speedup 1.0000x reference)
import jax
import jax.numpy as jnp
from jax.experimental import pallas as pl


def kernel(input_ids, attention_mask, embedding_table):
    raise NotImplementedError("write your pallas kernel here")


# speedup vs baseline: 2.0429x; 2.0429x over previous
import functools

import jax
import jax.numpy as jnp
from jax import lax
from jax.experimental import pallas as pl
from jax.experimental.pallas import tpu as pltpu
from jax.experimental.pallas import tpu_sc as plsc

BATCH, SEQ, VOCAB, DIM = 4096, 200, 1000000, 32
NC, NS = 2, 16
NW = NC * NS
RPW = BATCH // NW
G = 16
LANES = 16


def _body(ids_hbm, mask_hbm, table_hbm, out_hbm, ids_v, mask_v, rows_v,
          res_v, sem):
    wid = lax.axis_index("s") * NC + lax.axis_index("c")
    base = wid * RPW

    def group(gi, _):
        g0 = base + gi * G
        pltpu.sync_copy(ids_hbm.at[pl.ds(g0, G)], ids_v)
        pltpu.sync_copy(mask_hbm.at[pl.ds(g0, G)], mask_v)

        def row(j, _):
            cp0 = pltpu.async_copy(
                table_hbm.at[ids_v.at[j, pl.ds(0, 128)]],
                rows_v.at[pl.ds(0, 128)], sem)
            cp1 = pltpu.async_copy(
                table_hbm.at[ids_v.at[j, pl.ds(128, 72)]],
                rows_v.at[pl.ds(128, 72)], sem)
            cp0.wait()
            cp1.wait()

            def chunk(c, carry):
                a0, a1, cf = carry
                mv = mask_v[j, pl.ds(c * LANES, LANES)].astype(jnp.float32)
                for t in range(LANES):
                    s = c * LANES + t
                    mf = jnp.full((LANES,), mv[t])
                    a0 = a0 + rows_v[s, pl.ds(0, LANES)] * mf
                    a1 = a1 + rows_v[s, pl.ds(LANES, LANES)] * mf
                    cf = cf + mf
                return a0, a1, cf

            zero = jnp.zeros((LANES,), jnp.float32)
            a0, a1, cf = lax.fori_loop(0, SEQ // LANES, chunk,
                                       (zero, zero, zero))
            mv = mask_v[j, pl.ds(SEQ - LANES, LANES)].astype(jnp.float32)
            for t in range(LANES // 2, LANES):
                s = SEQ - LANES + t
                mf = jnp.full((LANES,), mv[t])
                a0 = a0 + rows_v[s, pl.ds(0, LANES)] * mf
                a1 = a1 + rows_v[s, pl.ds(LANES, LANES)] * mf
                cf = cf + mf
            inv = 1.0 / cf
            r = gi * G + j
            res_v[r, pl.ds(0, LANES)] = a0 * inv
            res_v[r, pl.ds(LANES, LANES)] = a1 * inv
            return 0

        lax.fori_loop(0, G, row, 0)
        return 0

    lax.fori_loop(0, RPW // G, group, 0)
    pltpu.sync_copy(res_v, out_hbm.at[pl.ds(base, RPW)])


@functools.partial(
    pl.kernel,
    out_type=jax.ShapeDtypeStruct((BATCH, DIM), jnp.float32),
    mesh=plsc.VectorSubcoreMesh(core_axis_name="c", subcore_axis_name="s",
                                num_cores=NC, num_subcores=NS),
    compiler_params=pltpu.CompilerParams(use_tc_tiling_on_sc=False),
    scratch_types=[
        pltpu.VMEM((G, SEQ), jnp.int32),
        pltpu.VMEM((G, SEQ), jnp.int32),
        pltpu.VMEM((SEQ, DIM), jnp.float32),
        pltpu.VMEM((RPW, DIM), jnp.float32),
        pltpu.SemaphoreType.DMA,
    ],
)
def _encode(ids_hbm, mask_hbm, table_hbm, out_hbm, ids_v, mask_v, rows_v,
            res_v, sem):
    _body(ids_hbm, mask_hbm, table_hbm, out_hbm, ids_v, mask_v, rows_v,
          res_v, sem)


def kernel(input_ids, attention_mask, embedding_table):
    return _encode(input_ids, attention_mask, embedding_table)


# speedup vs baseline: 2.4539x; 1.2011x over previous
import functools

import jax
import jax.numpy as jnp
from jax import lax
from jax.experimental import pallas as pl
from jax.experimental.pallas import tpu as pltpu
from jax.experimental.pallas import tpu_sc as plsc

BATCH, SEQ, VOCAB, DIM = 4096, 200, 1000000, 32
NC, NS = 2, 16
NW = NC * NS
RPW = BATCH // NW
LANES = 16
NBUF = 4
SPLIT = 128
REST = SEQ - SPLIT


def _fire_gather(table_hbm, ids_v, buf, j, sem):
    c0 = pltpu.async_copy(table_hbm.at[ids_v.at[j, pl.ds(0, SPLIT)]],
                          buf.at[pl.ds(0, SPLIT)], sem)
    c1 = pltpu.async_copy(table_hbm.at[ids_v.at[j, pl.ds(SPLIT, REST)]],
                          buf.at[pl.ds(SPLIT, REST)], sem)
    return c0, c1


def _body(ids_hbm, mask_hbm, table_hbm, out_hbm, ids_v, mask_v, rows_v,
          res_v, *sems):
    wid = lax.axis_index("s") * NC + lax.axis_index("c")
    base = wid * RPW

    pltpu.sync_copy(ids_hbm.at[pl.ds(base, RPW)], ids_v)
    pltpu.sync_copy(mask_hbm.at[pl.ds(base, RPW)], mask_v)

    for b in range(NBUF):
        _fire_gather(table_hbm, ids_v, rows_v.at[b], b, sems[b])

    def pool_row(b, j):
        def chunk(c, carry):
            a0, a1, cf = carry
            mv = mask_v[j, pl.ds(c * LANES, LANES)].astype(jnp.float32)
            for t in range(LANES):
                s = c * LANES + t
                mf = jnp.full((LANES,), mv[t])
                a0 = a0 + rows_v[b, s, pl.ds(0, LANES)] * mf
                a1 = a1 + rows_v[b, s, pl.ds(LANES, LANES)] * mf
                cf = cf + mf
            return a0, a1, cf

        zero = jnp.zeros((LANES,), jnp.float32)
        a0, a1, cf = lax.fori_loop(0, SEQ // LANES, chunk,
                                   (zero, zero, zero))
        mv = mask_v[j, pl.ds(SEQ - LANES, LANES)].astype(jnp.float32)
        for t in range(LANES // 2, LANES):
            s = SEQ - LANES + t
            mf = jnp.full((LANES,), mv[t])
            a0 = a0 + rows_v[b, s, pl.ds(0, LANES)] * mf
            a1 = a1 + rows_v[b, s, pl.ds(LANES, LANES)] * mf
            cf = cf + mf
        inv = 1.0 / cf
        res_v[j, pl.ds(0, LANES)] = a0 * inv
        res_v[j, pl.ds(LANES, LANES)] = a1 * inv

    def quad(q, _):
        for b in range(NBUF):
            j = q * NBUF + b
            pltpu.make_async_copy(
                table_hbm.at[ids_v.at[j, pl.ds(0, SPLIT)]],
                rows_v.at[b, pl.ds(0, SPLIT)], sems[b]).wait()
            pltpu.make_async_copy(
                table_hbm.at[ids_v.at[j, pl.ds(SPLIT, REST)]],
                rows_v.at[b, pl.ds(SPLIT, REST)], sems[b]).wait()
            pool_row(b, j)
            nj = j + NBUF

            @pl.when(nj < RPW)
            def _():
                _fire_gather(table_hbm, ids_v, rows_v.at[b], nj, sems[b])
        return 0

    lax.fori_loop(0, RPW // NBUF, quad, 0)
    pltpu.sync_copy(res_v, out_hbm.at[pl.ds(base, RPW)])


@functools.partial(
    pl.kernel,
    out_type=jax.ShapeDtypeStruct((BATCH, DIM), jnp.float32),
    mesh=plsc.VectorSubcoreMesh(core_axis_name="c", subcore_axis_name="s",
                                num_cores=NC, num_subcores=NS),
    compiler_params=pltpu.CompilerParams(use_tc_tiling_on_sc=False),
    scratch_types=[
        pltpu.VMEM((RPW, SEQ), jnp.int32),
        pltpu.VMEM((RPW, SEQ), jnp.int32),
        pltpu.VMEM((NBUF, SEQ, DIM), jnp.float32),
        pltpu.VMEM((RPW, DIM), jnp.float32),
    ] + [pltpu.SemaphoreType.DMA] * NBUF,
)
def _encode(ids_hbm, mask_hbm, table_hbm, out_hbm, ids_v, mask_v, rows_v,
            res_v, *sems):
    _body(ids_hbm, mask_hbm, table_hbm, out_hbm, ids_v, mask_v, rows_v,
          res_v, *sems)


def kernel(input_ids, attention_mask, embedding_table):
    return _encode(input_ids, attention_mask, embedding_table)
